# baseline (speedup 1.0000x reference)
import dataclasses
import functools

import jax
import jax.numpy as jnp
from jax import lax
from jax.experimental import pallas as pl
from jax.experimental.pallas import tpu as pltpu
from jax.experimental.pallas import tpu_sc as plsc

B = 1024
K = 512
D = 128
LANES = 16
NW = 32
B_PER_W = B // NW
KCH = 128
NKCH = K // KCH
NCH = D // LANES


def _q_body(v_ref, wt_ref, q_ref):
    q_ref[...] = jnp.dot(v_ref[...], wt_ref[...],
                         preferred_element_type=jnp.float32)


def _pack_half(u):
    return (((u[:, :D // 2] + 0x8000) >> 16)
            | ((u[:, D // 2:] + 0x8000) & jnp.uint32(0xFFFF0000)))


def _pack_body(t_ref, b_ref, o_ref):
    pt = _pack_half(lax.bitcast_convert_type(t_ref[...], jnp.uint32))
    pb = _pack_half(lax.bitcast_convert_type(b_ref[...], jnp.uint32))
    o_ref[...] = lax.bitcast_convert_type(
        jnp.concatenate([pt, pb], axis=1), jnp.int32)


def _pack_table(table):
    nv = table.shape[0]
    half = nv // 2
    grid = 5
    g = half // grid
    packed2 = pl.pallas_call(
        _pack_body,
        grid=(grid,),
        in_specs=[pl.BlockSpec((g, D), lambda i: (i, 0)),
                  pl.BlockSpec((g, D), lambda i: (i + grid, 0))],
        out_specs=pl.BlockSpec((g, D), lambda i: (i, 0)),
        out_shape=jax.ShapeDtypeStruct((half, D), jnp.int32),
    )(table, table)
    return packed2.reshape(nv, D // 2)


NCHUNKS = B_PER_W * NKCH
NBUF = 4


def _sc_scores(q, impl_sets3, table):
    mesh = plsc.VectorSubcoreMesh(core_axis_name="c", subcore_axis_name="s")
    cp = pltpu.CompilerParams()
    for fld, val in (("needs_layout_passes", False),
                     ("use_tc_tiling_on_sc", False)):
        if fld in pltpu.CompilerParams.__dataclass_fields__:
            cp = dataclasses.replace(cp, **{fld: val})

    @functools.partial(
        pl.kernel,
        out_type=jax.ShapeDtypeStruct((B, K), jnp.float32),
        mesh=mesh,
        compiler_params=cp,
        scratch_types=[
            pltpu.VMEM((NCHUNKS, KCH), jnp.int32),
            pltpu.VMEM((B_PER_W, D // 2), jnp.int32),
            pltpu.VMEM((B_PER_W, K), jnp.float32),
            pltpu.VMEM((2, LANES, LANES + 1), jnp.float32),
        ]
        + [pltpu.VMEM((KCH, D // 2), jnp.int32) for _ in range(NBUF)]
        + [pltpu.SemaphoreType.DMA for _ in range(NBUF)],
    )
    def k(q_hbm, idx_hbm, table_hbm, out_hbm, idx_v, q_v, s_v, p_v, *bufs_sems):
        bufs = bufs_sems[:NBUF]
        sems = bufs_sems[NBUF:]
        wid = lax.axis_index("s") * 2 + lax.axis_index("c")
        row_ids = lax.iota(jnp.int32, LANES)

        pltpu.sync_copy(idx_hbm.at[wid], idx_v)
        pltpu.sync_copy(q_hbm.at[pl.ds(wid * B_PER_W, B_PER_W)], q_v)

        def start(t, i):
            pltpu.async_copy(table_hbm.at[idx_v.at[t]], bufs[i], sems[i])

        def wait(i):
            pltpu.make_async_copy(
                table_hbm.at[pl.ds(0, KCH)], bufs[i], sems[i]).wait()

        for i in range(NBUF):
            start(i, i)

        @pl.loop(0, NCHUNKS, step=NBUF)
        def _(t0):
            for i in range(NBUF):
                t = t0 + i
                bl = t // NKCH
                col0 = (t % NKCH) * KCH
                wait(i)
                qs = [plsc.bitcast(q_v[bl, pl.ds(c * LANES, LANES)],
                                   jnp.bfloat16)
                      for c in range(NCH // 2)]

                def row_chunk(r, c):
                    return plsc.bitcast(
                        bufs[i][r, pl.ds(c * LANES, LANES)], jnp.bfloat16)

                def fma_partials(r0, bank):
                    accs = [row_chunk(r0 + j, 0) * qs[0]
                            for j in range(LANES)]
                    for c in range(1, NCH // 2):
                        for j in range(LANES):
                            accs[j] = accs[j] + row_chunk(r0 + j, c) * qs[c]
                    for j in range(LANES):
                        lo, hi = plsc.unpack(
                            accs[j], format=plsc.PackFormat.INTERLEAVED)
                        p_v[bank, j, pl.ds(0, LANES)] = lo + hi

                def reduce_store(r0, bank):
                    g = [plsc.load_gather(
                            p_v.at[bank],
                            [row_ids, jnp.full((LANES,), l, jnp.int32)])
                         for l in range(LANES)]
                    while len(g) > 1:
                        g = [g[m] + g[m + 1] for m in range(0, len(g), 2)]
                    s_v[bl, pl.ds(col0 + r0, LANES)] = g[0]

                @pl.loop(0, KCH, step=2 * LANES)
                def _(r0):
                    fma_partials(r0, 0)
                    fma_partials(r0 + LANES, 1)
                    reduce_store(r0, 0)
                    reduce_store(r0 + LANES, 1)

                @pl.when(t + NBUF < NCHUNKS)
                def _():
                    start(t + NBUF, i)

        pltpu.sync_copy(s_v, out_hbm.at[pl.ds(wid * B_PER_W, B_PER_W)])

    return k(q, impl_sets3, table)


def kernel(vectors, impl_sets, table, W):
    q = pl.pallas_call(
        _q_body,
        out_shape=jax.ShapeDtypeStruct((B, D), jnp.float32),
    )(vectors, W.T)

    def pack_bf16(x):
        lo = lax.bitcast_convert_type(
            x[..., :D // 2].astype(jnp.bfloat16), jnp.uint16).astype(jnp.uint32)
        hi = lax.bitcast_convert_type(
            x[..., D // 2:].astype(jnp.bfloat16), jnp.uint16).astype(jnp.uint32)
        return lax.bitcast_convert_type(lo | (hi << 16), jnp.int32)

    half = table.shape[0] // 2
    idx = jnp.where(impl_sets < half, 2 * impl_sets,
                    2 * impl_sets - (2 * half - 1)).astype(jnp.int32)
    scores = _sc_scores(pack_bf16(q),
                        idx.reshape(NW, NCHUNKS, KCH),
                        _pack_table(table))
    return (impl_sets, scores)


# speedup vs baseline: 1.0138x; 1.0138x over previous
import dataclasses
import functools

import jax
import jax.numpy as jnp
from jax import lax
from jax.experimental import pallas as pl
from jax.experimental.pallas import tpu as pltpu
from jax.experimental.pallas import tpu_sc as plsc

B = 1024
K = 512
D = 128
LANES = 16
NW = 32
B_PER_W = B // NW
KCH = 128
NKCH = K // KCH
NCH = D // LANES


def _q_body(v_ref, wt_ref, q_ref):
    q_ref[...] = jnp.dot(v_ref[...], wt_ref[...],
                         preferred_element_type=jnp.float32)


def _pack_half(u):
    return ((u[:, :D // 2] >> 16)
            | (u[:, D // 2:] & jnp.uint32(0xFFFF0000)))


def _pack_body(t_ref, b_ref, o_ref):
    pt = _pack_half(lax.bitcast_convert_type(t_ref[...], jnp.uint32))
    pb = _pack_half(lax.bitcast_convert_type(b_ref[...], jnp.uint32))
    o_ref[...] = lax.bitcast_convert_type(
        jnp.concatenate([pt, pb], axis=1), jnp.int32)


def _pack_table(table):
    nv = table.shape[0]
    half = nv // 2
    grid = 10
    g = half // grid
    packed2 = pl.pallas_call(
        _pack_body,
        grid=(grid,),
        in_specs=[pl.BlockSpec((g, D), lambda i: (i, 0)),
                  pl.BlockSpec((g, D), lambda i: (i + grid, 0))],
        out_specs=pl.BlockSpec((g, D), lambda i: (i, 0)),
        out_shape=jax.ShapeDtypeStruct((half, D), jnp.int32),
    )(table, table)
    return packed2.reshape(nv, D // 2)


NCHUNKS = B_PER_W * NKCH
NBUF = 4


def _sc_scores(q, impl_sets3, table):
    mesh = plsc.VectorSubcoreMesh(core_axis_name="c", subcore_axis_name="s")
    cp = pltpu.CompilerParams()
    for fld, val in (("needs_layout_passes", False),
                     ("use_tc_tiling_on_sc", False)):
        if fld in pltpu.CompilerParams.__dataclass_fields__:
            cp = dataclasses.replace(cp, **{fld: val})

    @functools.partial(
        pl.kernel,
        out_type=jax.ShapeDtypeStruct((B, K), jnp.float32),
        mesh=mesh,
        compiler_params=cp,
        scratch_types=[
            pltpu.VMEM((NCHUNKS, KCH), jnp.int32),
            pltpu.VMEM((B_PER_W, D // 2), jnp.int32),
            pltpu.VMEM((B_PER_W, K), jnp.float32),
            pltpu.VMEM((2, LANES, LANES + 1), jnp.float32),
        ]
        + [pltpu.VMEM((KCH, D // 2), jnp.int32) for _ in range(NBUF)]
        + [pltpu.SemaphoreType.DMA for _ in range(NBUF)],
    )
    def k(q_hbm, idx_hbm, table_hbm, out_hbm, idx_v, q_v, s_v, p_v, *bufs_sems):
        bufs = bufs_sems[:NBUF]
        sems = bufs_sems[NBUF:]
        wid = lax.axis_index("s") * 2 + lax.axis_index("c")
        row_ids = lax.iota(jnp.int32, LANES)

        pltpu.sync_copy(idx_hbm.at[wid], idx_v)
        pltpu.sync_copy(q_hbm.at[pl.ds(wid * B_PER_W, B_PER_W)], q_v)

        def start(t, i):
            pltpu.async_copy(table_hbm.at[idx_v.at[t]], bufs[i], sems[i])

        def wait(i):
            pltpu.make_async_copy(
                table_hbm.at[pl.ds(0, KCH)], bufs[i], sems[i]).wait()

        for i in range(NBUF):
            start(i, i)

        @pl.loop(0, NCHUNKS, step=NBUF)
        def _(t0):
            for i in range(NBUF):
                t = t0 + i
                bl = t // NKCH
                col0 = (t % NKCH) * KCH
                wait(i)
                qs = [plsc.bitcast(q_v[bl, pl.ds(c * LANES, LANES)],
                                   jnp.bfloat16)
                      for c in range(NCH // 2)]

                def row_chunk(r, c):
                    return plsc.bitcast(
                        bufs[i][r, pl.ds(c * LANES, LANES)], jnp.bfloat16)

                def fma_partials(r0, bank):
                    accs = [row_chunk(r0 + j, 0) * qs[0]
                            for j in range(LANES)]
                    for c in range(1, NCH // 2):
                        for j in range(LANES):
                            accs[j] = accs[j] + row_chunk(r0 + j, c) * qs[c]
                    for j in range(LANES):
                        lo, hi = plsc.unpack(
                            accs[j], format=plsc.PackFormat.INTERLEAVED)
                        p_v[bank, j, pl.ds(0, LANES)] = lo + hi

                def reduce_store(r0, bank):
                    g = [plsc.load_gather(
                            p_v.at[bank],
                            [row_ids, jnp.full((LANES,), l, jnp.int32)])
                         for l in range(LANES)]
                    while len(g) > 1:
                        g = [g[m] + g[m + 1] for m in range(0, len(g), 2)]
                    s_v[bl, pl.ds(col0 + r0, LANES)] = g[0]

                @pl.loop(0, KCH, step=2 * LANES)
                def _(r0):
                    fma_partials(r0, 0)
                    fma_partials(r0 + LANES, 1)
                    reduce_store(r0, 0)
                    reduce_store(r0 + LANES, 1)

                @pl.when(t + NBUF < NCHUNKS)
                def _():
                    start(t + NBUF, i)

        pltpu.sync_copy(s_v, out_hbm.at[pl.ds(wid * B_PER_W, B_PER_W)])

    return k(q, impl_sets3, table)


def kernel(vectors, impl_sets, table, W):
    q = pl.pallas_call(
        _q_body,
        out_shape=jax.ShapeDtypeStruct((B, D), jnp.float32),
    )(vectors, W.T)

    def pack_bf16(x):
        lo = lax.bitcast_convert_type(
            x[..., :D // 2].astype(jnp.bfloat16), jnp.uint16).astype(jnp.uint32)
        hi = lax.bitcast_convert_type(
            x[..., D // 2:].astype(jnp.bfloat16), jnp.uint16).astype(jnp.uint32)
        return lax.bitcast_convert_type(lo | (hi << 16), jnp.int32)

    half = table.shape[0] // 2
    idx = jnp.where(impl_sets < half, 2 * impl_sets,
                    2 * impl_sets - (2 * half - 1)).astype(jnp.int32)
    scores = _sc_scores(pack_bf16(q),
                        idx.reshape(NW, NCHUNKS, KCH),
                        _pack_table(table))
    return (impl_sets, scores)
